# Initial kernel scaffold; baseline (speedup 1.0000x reference)
import jax
import jax.numpy as jnp
from jax.experimental import pallas as pl


def kernel(x, A, B):
    raise NotImplementedError("write your pallas kernel here")


# speedup vs baseline: 1.5259x; 1.5259x over previous
import functools

import jax
import jax.numpy as jnp
from jax import lax
from jax.experimental import pallas as pl
from jax.experimental.pallas import tpu as pltpu
from jax.experimental.pallas import tpu_sc as plsc

_R = 4
_RP = 8
_D = 64
_SCALING = 1.0 / _R

_NC = 2
_NS = 16
_NW = _NC * _NS

_CHUNK = 128
_FIRE = 8


def _gather_kernel(n_tokens: int):
    b_per_w = n_tokens // _NW
    n_chunks = b_per_w // _CHUNK
    mesh = plsc.VectorSubcoreMesh(core_axis_name="c", subcore_axis_name="s")

    @functools.partial(
        pl.kernel,
        mesh=mesh,
        out_type=jax.ShapeDtypeStruct((n_tokens, _RP), jnp.float32),
        scratch_types=[
            pltpu.VMEM((n_chunks, _CHUNK), jnp.int32),
            pltpu.VMEM((b_per_w, _RP), jnp.float32),
            pltpu.SemaphoreType.DMA,
        ],
        compiler_params=pltpu.CompilerParams(use_tc_tiling_on_sc=False),
    )
    def gather(table_hbm, idx_hbm, out_hbm, idx_v, rows_v, sem):
        wid = lax.axis_index("s") * _NC + lax.axis_index("c")
        base = wid * b_per_w
        pltpu.sync_copy(idx_hbm.at[wid], idx_v)

        def copy(j):
            return pltpu.make_async_copy(
                table_hbm.at[idx_v.at[j]],
                rows_v.at[pl.ds(j * _CHUNK, _CHUNK)],
                sem,
            )

        def body(j, carry):
            copy(j).start()

            @pl.when(j >= _FIRE)
            def _():
                copy(j - _FIRE).wait()

            return carry

        lax.fori_loop(0, n_chunks, body, 0, unroll=False)
        for j in range(max(n_chunks - _FIRE, 0), n_chunks):
            copy(j).wait()
        pltpu.sync_copy(rows_v, out_hbm.at[pl.ds(base, b_per_w)])

    return gather


def _matmul_call(emb, bt, n_tokens: int):
    bm = 4096

    def body(e_ref, bt_ref, o_ref):
        o_ref[...] = (
            jnp.dot(e_ref[...], bt_ref[...], preferred_element_type=jnp.float32)
            * _SCALING
        )

    return pl.pallas_call(
        body,
        grid=(n_tokens // bm,),
        in_specs=[
            pl.BlockSpec((bm, _RP), lambda i: (i, 0)),
            pl.BlockSpec((_RP, _D), lambda i: (0, 0)),
        ],
        out_specs=pl.BlockSpec((bm, _D), lambda i: (i, 0)),
        out_shape=jax.ShapeDtypeStruct((n_tokens, _D), jnp.float32),
    )(emb, bt)


def kernel(x, A, B):
    b, l = x.shape
    n_tokens = b * l
    idx = x.reshape(_NW, n_tokens // (_NW * _CHUNK), _CHUNK).astype(jnp.int32)
    v = A.shape[1]
    table = jnp.zeros((v, _RP), jnp.float32).at[:, :_R].set(A.T)
    bt = jnp.zeros((_RP, _D), jnp.float32).at[:_R, :].set(B.T)
    emb = _gather_kernel(n_tokens)(table, idx)
    out = _matmul_call(emb, bt, n_tokens)
    return out.reshape(b, l, _D)


# speedup vs baseline: 6.9126x; 4.5302x over previous
import functools

import jax
import jax.numpy as jnp
from jax import lax
from jax.experimental import pallas as pl
from jax.experimental.pallas import tpu as pltpu
from jax.experimental.pallas import tpu_sc as plsc

_R = 4
_RP = 8
_D = 64
_SCALING = 1.0 / _R

_NC = 2
_NS = 16
_NW = _NC * _NS

_CHUNK = 128
_FIRE = 8
_LANES = 16
_CV = 8000
_TPR = 128 // _RP


def _transpose_kernel(v: int):
    n_chunks = v // _CV
    mesh = plsc.VectorSubcoreMesh(core_axis_name="c", subcore_axis_name="s")

    @functools.partial(
        pl.kernel,
        mesh=mesh,
        out_type=jax.ShapeDtypeStruct((v * _RP,), jnp.float32),
        scratch_types=[
            pltpu.VMEM((_R, _CV), jnp.float32),
            pltpu.VMEM((_CV * _RP,), jnp.float32),
            pltpu.SemaphoreType.DMA,
        ],
        compiler_params=pltpu.CompilerParams(
            use_tc_tiling_on_sc=False, needs_layout_passes=False
        ),
    )
    def transpose(a_hbm, table_hbm, buf_v, slab_v, sem):
        wid = lax.axis_index("s") * _NC + lax.axis_index("c")
        io8 = lax.broadcasted_iota(jnp.int32, (_LANES,), 0) * _RP

        zvec = jnp.zeros((_LANES,), jnp.float32)

        def zbody(k, carry):
            slab_v[pl.ds(k * _LANES, _LANES)] = zvec
            return carry

        lax.fori_loop(0, _CV * _RP // _LANES, zbody, 0, unroll=False)

        def do_chunk(c):
            for r in range(_R):
                pltpu.make_async_copy(
                    a_hbm.at[pl.ds(r * v + c * _CV, _CV)], buf_v.at[r], sem
                ).start()
            for r in range(_R):
                pltpu.make_async_copy(
                    a_hbm.at[pl.ds(0, _CV)], buf_v.at[0], sem
                ).wait()
            def qbody(q, carry):
                for r in range(_R):
                    plsc.store_scatter(
                        slab_v,
                        [io8 + (q * _LANES * _RP + r)],
                        buf_v[r, pl.ds(q * _LANES, _LANES)],
                    )
                return carry

            lax.fori_loop(0, _CV // _LANES, qbody, 0, unroll=False)
            pltpu.sync_copy(
                slab_v, table_hbm.at[pl.ds(c * _CV * _RP, _CV * _RP)]
            )

        for step in range(-(-n_chunks // _NW)):
            c = wid + step * _NW

            @pl.when(c < n_chunks)
            def _():
                do_chunk(c)

    return transpose


def _gather_kernel(n_tokens: int, v: int):
    b_per_w = n_tokens // _NW
    n_chunks = b_per_w // _CHUNK
    mesh = plsc.VectorSubcoreMesh(core_axis_name="c", subcore_axis_name="s")

    @functools.partial(
        pl.kernel,
        mesh=mesh,
        out_type=jax.ShapeDtypeStruct((n_tokens, _RP), jnp.float32),
        scratch_types=[
            pltpu.VMEM((b_per_w,), jnp.int32),
            pltpu.VMEM((b_per_w, _RP), jnp.float32),
            pltpu.SemaphoreType.DMA,
        ],
        compiler_params=pltpu.CompilerParams(
            use_tc_tiling_on_sc=False, needs_layout_passes=False
        ),
    )
    def gather(table_hbm, idx_hbm, out_hbm, idx_v, rows_v, sem):
        wid = lax.axis_index("s") * _NC + lax.axis_index("c")
        base = wid * b_per_w
        pltpu.sync_copy(idx_hbm.at[pl.ds(base, b_per_w)], idx_v)

        def copy(j):
            return pltpu.make_async_copy(
                table_hbm.at[idx_v.at[pl.ds(j * _CHUNK, _CHUNK)]],
                rows_v.at[pl.ds(j * _CHUNK, _CHUNK)],
                sem,
            )

        def body(j, carry):
            copy(j).start()

            @pl.when(j >= _FIRE)
            def _():
                copy(j - _FIRE).wait()

            return carry

        lax.fori_loop(0, n_chunks, body, 0, unroll=False)
        for j in range(max(n_chunks - _FIRE, 0), n_chunks):
            copy(j).wait()
        pltpu.sync_copy(rows_v, out_hbm.at[pl.ds(base, b_per_w)])

    return gather


def _matmul_call(emb, bt8, n_tokens: int):
    bm = 4096

    def body(e_ref, bt_ref, o_ref):
        o_ref[...] = jnp.dot(
            e_ref[...], bt_ref[...], preferred_element_type=jnp.float32
        )

    return pl.pallas_call(
        body,
        grid=(n_tokens // bm,),
        in_specs=[
            pl.BlockSpec((bm, _RP), lambda i: (i, 0)),
            pl.BlockSpec((_RP, _D), lambda i: (0, 0)),
        ],
        out_specs=pl.BlockSpec((bm, _D), lambda i: (i, 0)),
        out_shape=jax.ShapeDtypeStruct((n_tokens, _D), jnp.float32),
    )(emb, bt8)


def kernel(x, A, B):
    b, l = x.shape
    n_tokens = b * l
    v = A.shape[1]
    idx = x.reshape(n_tokens).astype(jnp.int32)
    a_flat = A.reshape(_R * v)
    bt8 = jnp.zeros((_RP, _D), jnp.float32).at[:_R, :].set(B.T * _SCALING)

    table = _transpose_kernel(v)(a_flat).reshape(v, _RP)
    emb = _gather_kernel(n_tokens, v)(table, idx)
    out = _matmul_call(emb, bt8, n_tokens)
    return out.reshape(b, l, _D)


# speedup vs baseline: 10.3471x; 1.4968x over previous
import functools

import jax
import jax.numpy as jnp
from jax import lax
from jax.experimental import pallas as pl
from jax.experimental.pallas import tpu as pltpu
from jax.experimental.pallas import tpu_sc as plsc

_R = 4
_RP = 8
_D = 64
_SCALING = 1.0 / _R

_NC = 2
_NS = 16
_NW = _NC * _NS

_CHUNK = 128
_FIRE = 8
_LANES = 16
_CV = 8000
_TPR = 128 // _RP


def _transpose_kernel(v: int):
    n_chunks = v // _CV
    mesh = plsc.VectorSubcoreMesh(core_axis_name="c", subcore_axis_name="s")

    @functools.partial(
        pl.kernel,
        mesh=mesh,
        out_type=jax.ShapeDtypeStruct((v * _RP,), jnp.float32),
        scratch_types=[
            pltpu.VMEM((_R, _CV), jnp.float32),
            pltpu.VMEM((_CV * _RP,), jnp.float32),
            pltpu.SemaphoreType.DMA,
        ],
        compiler_params=pltpu.CompilerParams(
            use_tc_tiling_on_sc=False, needs_layout_passes=False
        ),
    )
    def transpose(a_hbm, table_hbm, buf_v, slab_v, sem):
        wid = lax.axis_index("s") * _NC + lax.axis_index("c")
        io8 = lax.broadcasted_iota(jnp.int32, (_LANES,), 0) * _RP

        zvec = jnp.zeros((_LANES,), jnp.float32)

        def zbody(k, carry):
            slab_v[pl.ds(k * _LANES, _LANES)] = zvec
            return carry

        lax.fori_loop(0, _CV * _RP // _LANES, zbody, 0, unroll=False)

        def do_chunk(c):
            for r in range(_R):
                pltpu.make_async_copy(
                    a_hbm.at[pl.ds(r * v + c * _CV, _CV)], buf_v.at[r], sem
                ).start()
            for r in range(_R):
                pltpu.make_async_copy(
                    a_hbm.at[pl.ds(0, _CV)], buf_v.at[0], sem
                ).wait()
            def qbody(q, carry):
                for r in range(_R):
                    plsc.store_scatter(
                        slab_v,
                        [io8 + (q * _LANES * _RP + r)],
                        buf_v[r, pl.ds(q * _LANES, _LANES)],
                    )
                return carry

            lax.fori_loop(0, _CV // _LANES, qbody, 0, unroll=False)
            pltpu.sync_copy(
                slab_v, table_hbm.at[pl.ds(c * _CV * _RP, _CV * _RP)]
            )

        for step in range(-(-n_chunks // _NW)):
            c = wid + step * _NW

            @pl.when(c < n_chunks)
            def _():
                do_chunk(c)

    return transpose


def _gather_kernel(n_tokens: int, v: int):
    b_per_w = n_tokens // _NW
    n_chunks = b_per_w // _CHUNK
    mesh = plsc.VectorSubcoreMesh(core_axis_name="c", subcore_axis_name="s")

    @functools.partial(
        pl.kernel,
        mesh=mesh,
        out_type=jax.ShapeDtypeStruct((_NW, n_tokens // _NW, _RP), jnp.float32),
        scratch_types=[
            pltpu.VMEM((b_per_w,), jnp.int32),
            pltpu.VMEM((b_per_w, _RP), jnp.float32),
            pltpu.SemaphoreType.DMA,
        ],
        compiler_params=pltpu.CompilerParams(
            use_tc_tiling_on_sc=False, needs_layout_passes=False
        ),
    )
    def gather(table_hbm, idx_hbm, out_hbm, idx_v, rows_v, sem):
        wid = lax.axis_index("s") * _NC + lax.axis_index("c")
        base = wid * b_per_w
        pltpu.sync_copy(idx_hbm.at[pl.ds(base, b_per_w)], idx_v)

        def copy(j):
            return pltpu.make_async_copy(
                table_hbm.at[idx_v.at[pl.ds(j * _CHUNK, _CHUNK)]],
                rows_v.at[pl.ds(j * _CHUNK, _CHUNK)],
                sem,
            )

        def body(j, carry):
            copy(j).start()

            @pl.when(j >= _FIRE)
            def _():
                copy(j - _FIRE).wait()

            return carry

        lax.fori_loop(0, n_chunks, body, 0, unroll=False)
        for j in range(max(n_chunks - _FIRE, 0), n_chunks):
            copy(j).wait()
        pltpu.sync_copy(rows_v, out_hbm.at[wid])

    return gather


def _matmul_call(emb2, w, n_rows: int):
    bm = 1600
    n_cols = _TPR * _D

    def body(e_ref, w_ref, o_ref):
        o_ref[...] = jnp.dot(
            e_ref[...], w_ref[...], preferred_element_type=jnp.float32
        )

    return pl.pallas_call(
        body,
        grid=(n_rows // bm,),
        in_specs=[
            pl.BlockSpec((bm, 128), lambda i: (i, 0)),
            pl.BlockSpec((128, n_cols), lambda i: (0, 0)),
        ],
        out_specs=pl.BlockSpec((bm, n_cols), lambda i: (i, 0)),
        out_shape=jax.ShapeDtypeStruct((n_rows, n_cols), jnp.float32),
    )(emb2, w)


def kernel(x, A, B):
    b, l = x.shape
    n_tokens = b * l
    v = A.shape[1]
    idx = x.reshape(n_tokens).astype(jnp.int32)
    a_flat = A.reshape(_R * v)
    bt8 = jnp.zeros((_RP, _D), jnp.float32).at[:_R, :].set(B.T * _SCALING)
    w = jnp.kron(jnp.eye(_TPR, dtype=jnp.float32), bt8)

    table = _transpose_kernel(v)(a_flat).reshape(v, _RP)
    emb = _gather_kernel(n_tokens, v)(table, idx)
    n_rows = n_tokens * _RP // 128
    emb2 = emb.reshape(n_rows, 128)
    out = _matmul_call(emb2, w, n_rows)
    return out.reshape(b, l, _D)
